# baseline (speedup 1.0000x reference)
import functools

import jax
import jax.numpy as jnp
from jax import lax
from jax.experimental import pallas as pl
from jax.experimental.pallas import tpu as pltpu
from jax.experimental.pallas import tpu_sc as plsc

N_ROWS = 320000
COLS = 5
N_ELEMS = N_ROWS * COLS
NW = 32
ROWS_PER_TILE = N_ROWS // NW
ELEMS_PER_TILE = ROWS_PER_TILE * COLS
NCHUNK = 5
CHUNK_ROWS = ROWS_PER_TILE // NCHUNK
CHUNK_ELEMS = CHUNK_ROWS * COLS
GROUPS = CHUNK_ELEMS // 80

_mesh = plsc.VectorSubcoreMesh(core_axis_name="c", subcore_axis_name="s")


@functools.partial(
    pl.kernel,
    mesh=_mesh,
    compiler_params=pltpu.CompilerParams(needs_layout_passes=False),
    out_type=[
        jax.ShapeDtypeStruct((NW, 16), jnp.float32),
        jax.ShapeDtypeStruct((NW, 16), jnp.float32),
    ],
    scratch_types=[
        pltpu.VMEM((CHUNK_ELEMS,), jnp.float32),
        pltpu.VMEM((CHUNK_ELEMS,), jnp.float32),
        pltpu.VMEM((CHUNK_ELEMS,), jnp.float32),
        pltpu.VMEM((CHUNK_ELEMS,), jnp.float32),
        pltpu.VMEM((CHUNK_ROWS,), jnp.int32),
        pltpu.VMEM((CHUNK_ROWS,), jnp.int32),
        pltpu.VMEM((16,), jnp.float32),
        pltpu.VMEM((16,), jnp.float32),
        pltpu.SemaphoreType.DMA,
        pltpu.SemaphoreType.DMA,
    ],
)
def _sc_partials(pred_hbm, lab_hbm, tgt_hbm, out_loss, out_cnt,
                 pred_b0, pred_b1, tgt_b0, tgt_b1, lab_b0, lab_b1,
                 stage_l, stage_c, sem0, sem1):
    wid = lax.axis_index("s") * 2 + lax.axis_index("c")
    ebase = wid * ELEMS_PER_TILE
    rbase = wid * ROWS_PER_TILE
    pred_bufs = (pred_b0, pred_b1)
    tgt_bufs = (tgt_b0, tgt_b1)
    lab_bufs = (lab_b0, lab_b1)
    sems = (sem0, sem1)

    def issue(ci, b):
        eb = ebase + ci * CHUNK_ELEMS
        rb = rbase + ci * CHUNK_ROWS
        return (
            pltpu.async_copy(pred_hbm.at[pl.ds(eb, CHUNK_ELEMS)], pred_bufs[b], sems[b]),
            pltpu.async_copy(tgt_hbm.at[pl.ds(eb, CHUNK_ELEMS)], tgt_bufs[b], sems[b]),
            pltpu.async_copy(lab_hbm.at[pl.ds(rb, CHUNK_ROWS)], lab_bufs[b], sems[b]),
        )

    iota = lax.iota(jnp.int32, 16)
    pats = [(16 * r + iota) // 5 for r in range(COLS)]

    def compute_chunk(pb, tb, lb, acc, cnt):
        def cbody(i, cn):
            lv = lb[pl.ds(i * 16, 16)]
            return cn + jnp.where(lv == 1, 1.0, 0.0)

        cnt = lax.fori_loop(0, CHUNK_ROWS // 16, cbody, cnt)

        def gbody(g, ac):
            rowb = g * 16
            eb = g * 80
            for r in range(COLS):
                lab_g = plsc.load_gather(lb, [pats[r] + rowb])
                p = pb[pl.ds(eb + r * 16, 16)]
                t = tb[pl.ds(eb + r * 16, 16)]
                d = p - t
                ax = jnp.abs(d)
                c = jnp.minimum(ax, 1.0)
                h = 0.5 * c * c + (ax - c)
                ac = ac + jnp.where(lab_g == 1, h, 0.0)
            return ac

        acc = lax.fori_loop(0, GROUPS, gbody, acc)
        return acc, cnt

    pending = issue(0, 0)
    acc = jnp.zeros((16,), jnp.float32)
    cnt = jnp.zeros((16,), jnp.float32)
    for ci in range(NCHUNK):
        b = ci % 2
        nxt = issue(ci + 1, 1 - b) if ci + 1 < NCHUNK else None
        for h in pending:
            h.wait()
        pending = nxt
        acc, cnt = compute_chunk(pred_bufs[b], tgt_bufs[b], lab_bufs[b], acc, cnt)

    stage_l[...] = acc
    stage_c[...] = cnt
    pltpu.sync_copy(stage_l, out_loss.at[wid])
    pltpu.sync_copy(stage_c, out_cnt.at[wid])


def _fin_body(l_ref, c_ref, o_ref):
    s = jnp.sum(l_ref[...])
    c = jnp.sum(c_ref[...])
    o_ref[...] = jnp.reshape(s / (jnp.float32(COLS) * jnp.maximum(c, 1.0)), (1, 1))


_finalize = pl.pallas_call(
    _fin_body,
    out_shape=jax.ShapeDtypeStruct((1, 1), jnp.float32),
)


def kernel(out_ellipse, labels, ellipse_targets):
    pred = jnp.reshape(out_ellipse, (-1,))
    tgt = jnp.reshape(ellipse_targets, (-1,))
    lab = jnp.reshape(labels, (-1,))
    loss_p, cnt_p = _sc_partials(pred, lab, tgt)
    res = _finalize(loss_p, cnt_p)
    return jnp.reshape(res, ())


# speedup vs baseline: 9.4687x; 9.4687x over previous
import functools

import jax
import jax.numpy as jnp
from jax import lax
from jax.experimental import pallas as pl
from jax.experimental.pallas import tpu as pltpu
from jax.experimental.pallas import tpu_sc as plsc

N_ROWS = 320000
COLS = 5
NBLK = N_ROWS // 128
NW = 32
BLK_PER_TILE = NBLK // NW
EXTRA = NBLK - BLK_PER_TILE * NW
CHUNK_BLKS = 26
NCHUNK = BLK_PER_TILE // CHUNK_BLKS
CW = CHUNK_BLKS * 128

_mesh = plsc.VectorSubcoreMesh(core_axis_name="c", subcore_axis_name="s")


@functools.partial(
    pl.kernel,
    mesh=_mesh,
    compiler_params=pltpu.CompilerParams(needs_layout_passes=False),
    out_type=[
        jax.ShapeDtypeStruct((NW, 16), jnp.float32),
        jax.ShapeDtypeStruct((NW, 16), jnp.float32),
    ],
    scratch_types=[
        pltpu.VMEM((2, COLS, CW), jnp.float32),
        pltpu.VMEM((2, COLS, CW), jnp.float32),
        pltpu.VMEM((2, CW), jnp.int32),
        pltpu.VMEM((16,), jnp.float32),
        pltpu.VMEM((16,), jnp.float32),
        pltpu.SemaphoreType.DMA,
        pltpu.SemaphoreType.DMA,
    ],
)
def _sc_partials(pred_hbm, lab_hbm, tgt_hbm, out_loss, out_cnt,
                 pred_b, tgt_b, lab_b, stage_l, stage_c, sem0, sem1):
    wid = lax.axis_index("s") * 2 + lax.axis_index("c")
    base_row = wid * (BLK_PER_TILE * 128)
    sems = (sem0, sem1)

    def issue(ci, b):
        rb = base_row + ci * CW
        return [
            pltpu.async_copy(pred_hbm.at[:, pl.ds(rb, CW)], pred_b.at[b], sems[b]),
            pltpu.async_copy(tgt_hbm.at[:, pl.ds(rb, CW)], tgt_b.at[b], sems[b]),
            pltpu.async_copy(lab_hbm.at[pl.ds(rb, CW)], lab_b.at[b], sems[b]),
        ]

    def chunk_body(b, nvec, carry):
        def jbody(j, cr):
            ac, cn = cr
            o = j * 16
            lv = lab_b[b, pl.ds(o, 16)]
            m = lv == 1
            hsum = jnp.zeros((16,), jnp.float32)
            for c in range(COLS):
                p = pred_b[b, c, pl.ds(o, 16)]
                t = tgt_b[b, c, pl.ds(o, 16)]
                d = p - t
                ax = jnp.abs(d)
                mn = jnp.minimum(ax, 1.0)
                hsum = hsum + (0.5 * mn * mn + (ax - mn))
            ac = ac + jnp.where(m, hsum, 0.0)
            cn = cn + jnp.where(m, 1.0, 0.0)
            return (ac, cn)
        return lax.fori_loop(0, nvec, jbody, carry)

    acc = jnp.zeros((16,), jnp.float32)
    cnt = jnp.zeros((16,), jnp.float32)
    pending = issue(0, 0)
    for ci in range(NCHUNK):
        b = ci % 2
        nxt = issue(ci + 1, 1 - b) if ci + 1 < NCHUNK else None
        for h in pending:
            h.wait()
        pending = nxt
        acc, cnt = chunk_body(b, CW // 16, (acc, cnt))

    stage_l[...] = acc
    stage_c[...] = cnt

    @pl.when(wid < EXTRA)
    def _():
        rb = (NBLK - EXTRA) * 128 + wid * 128
        hs = [
            pltpu.async_copy(pred_hbm.at[:, pl.ds(rb, 128)],
                             pred_b.at[0, :, pl.ds(0, 128)], sem0),
            pltpu.async_copy(tgt_hbm.at[:, pl.ds(rb, 128)],
                             tgt_b.at[0, :, pl.ds(0, 128)], sem0),
            pltpu.async_copy(lab_hbm.at[pl.ds(rb, 128)],
                             lab_b.at[0, pl.ds(0, 128)], sem0),
        ]
        for h in hs:
            h.wait()
        a1, c1 = chunk_body(0, 8, (stage_l[...], stage_c[...]))
        stage_l[...] = a1
        stage_c[...] = c1

    pltpu.sync_copy(stage_l, out_loss.at[wid])
    pltpu.sync_copy(stage_c, out_cnt.at[wid])


def _fin_body(l_ref, c_ref, o_ref):
    s = jnp.sum(l_ref[...])
    c = jnp.sum(c_ref[...])
    o_ref[...] = jnp.reshape(s / (jnp.float32(COLS) * jnp.maximum(c, 1.0)), (1, 1))


_finalize = pl.pallas_call(
    _fin_body,
    out_shape=jax.ShapeDtypeStruct((1, 1), jnp.float32),
)


def kernel(out_ellipse, labels, ellipse_targets):
    pred_t = out_ellipse.T
    tgt_t = ellipse_targets.T
    lab = jnp.reshape(labels, (-1,))
    loss_p, cnt_p = _sc_partials(pred_t, lab, tgt_t)
    res = _finalize(loss_p, cnt_p)
    return jnp.reshape(res, ())


# speedup vs baseline: 9.4742x; 1.0006x over previous
import functools

import jax
import jax.numpy as jnp
from jax import lax
from jax.experimental import pallas as pl
from jax.experimental.pallas import tpu as pltpu
from jax.experimental.pallas import tpu_sc as plsc

N_ROWS = 320000
COLS = 5
NBLK = N_ROWS // 128
NW = 32
BLK_PER_TILE = NBLK // NW
EXTRA = NBLK - BLK_PER_TILE * NW
CHUNK_BLKS = 26
NCHUNK = BLK_PER_TILE // CHUNK_BLKS
CW = CHUNK_BLKS * 128

_mesh = plsc.VectorSubcoreMesh(core_axis_name="c", subcore_axis_name="s")


@functools.partial(
    pl.kernel,
    mesh=_mesh,
    compiler_params=pltpu.CompilerParams(
        needs_layout_passes=False, skip_device_barrier=True),
    out_type=[
        jax.ShapeDtypeStruct((NW, 16), jnp.float32),
        jax.ShapeDtypeStruct((NW, 16), jnp.float32),
    ],
    scratch_types=[
        pltpu.VMEM((2, COLS, CW), jnp.float32),
        pltpu.VMEM((2, COLS, CW), jnp.float32),
        pltpu.VMEM((2, CW), jnp.int32),
        pltpu.VMEM((16,), jnp.float32),
        pltpu.VMEM((16,), jnp.float32),
        pltpu.SemaphoreType.DMA,
        pltpu.SemaphoreType.DMA,
    ],
)
def _sc_partials(pred_hbm, lab_hbm, tgt_hbm, out_loss, out_cnt,
                 pred_b, tgt_b, lab_b, stage_l, stage_c, sem0, sem1):
    wid = lax.axis_index("s") * 2 + lax.axis_index("c")
    base_row = wid * (BLK_PER_TILE * 128)
    sems = (sem0, sem1)

    def issue(ci, b):
        rb = base_row + ci * CW
        return [
            pltpu.async_copy(pred_hbm.at[:, pl.ds(rb, CW)], pred_b.at[b], sems[b]),
            pltpu.async_copy(tgt_hbm.at[:, pl.ds(rb, CW)], tgt_b.at[b], sems[b]),
            pltpu.async_copy(lab_hbm.at[pl.ds(rb, CW)], lab_b.at[b], sems[b]),
        ]

    def chunk_body(b, nvec, carry):
        def jbody(j, cr):
            ac, cn = cr
            o = j * 16
            lv = lab_b[b, pl.ds(o, 16)]
            m = lv == 1
            hsum = jnp.zeros((16,), jnp.float32)
            for c in range(COLS):
                p = pred_b[b, c, pl.ds(o, 16)]
                t = tgt_b[b, c, pl.ds(o, 16)]
                d = p - t
                ax = jnp.abs(d)
                mn = jnp.minimum(ax, 1.0)
                hsum = hsum + (0.5 * mn * mn + (ax - mn))
            ac = ac + jnp.where(m, hsum, 0.0)
            cn = cn + jnp.where(m, 1.0, 0.0)
            return (ac, cn)
        return lax.fori_loop(0, nvec, jbody, carry)

    acc = jnp.zeros((16,), jnp.float32)
    cnt = jnp.zeros((16,), jnp.float32)
    pending = issue(0, 0)
    for ci in range(NCHUNK):
        b = ci % 2
        nxt = issue(ci + 1, 1 - b) if ci + 1 < NCHUNK else None
        for h in pending:
            h.wait()
        pending = nxt
        acc, cnt = chunk_body(b, CW // 16, (acc, cnt))

    stage_l[...] = acc
    stage_c[...] = cnt

    @pl.when(wid < EXTRA)
    def _():
        rb = (NBLK - EXTRA) * 128 + wid * 128
        hs = [
            pltpu.async_copy(pred_hbm.at[:, pl.ds(rb, 128)],
                             pred_b.at[0, :, pl.ds(0, 128)], sem0),
            pltpu.async_copy(tgt_hbm.at[:, pl.ds(rb, 128)],
                             tgt_b.at[0, :, pl.ds(0, 128)], sem0),
            pltpu.async_copy(lab_hbm.at[pl.ds(rb, 128)],
                             lab_b.at[0, pl.ds(0, 128)], sem0),
        ]
        for h in hs:
            h.wait()
        a1, c1 = chunk_body(0, 8, (stage_l[...], stage_c[...]))
        stage_l[...] = a1
        stage_c[...] = c1

    pltpu.sync_copy(stage_l, out_loss.at[wid])
    pltpu.sync_copy(stage_c, out_cnt.at[wid])


def _fin_body(l_ref, c_ref, o_ref):
    s = jnp.sum(l_ref[...])
    c = jnp.sum(c_ref[...])
    o_ref[...] = jnp.reshape(s / (jnp.float32(COLS) * jnp.maximum(c, 1.0)), (1, 1))


_finalize = pl.pallas_call(
    _fin_body,
    out_shape=jax.ShapeDtypeStruct((1, 1), jnp.float32),
)


def kernel(out_ellipse, labels, ellipse_targets):
    pred_t = out_ellipse.T
    tgt_t = ellipse_targets.T
    lab = jnp.reshape(labels, (-1,))
    loss_p, cnt_p = _sc_partials(pred_t, lab, tgt_t)
    res = _finalize(loss_p, cnt_p)
    return jnp.reshape(res, ())
